# baseline (speedup 1.0000x reference)
import jax
import jax.numpy as jnp
from jax import lax
from jax.experimental import pallas as pl
from jax.experimental.pallas import tpu as pltpu
from jax.experimental.pallas import tpu_sc as plsc

N_HALF = 13
N_TABLES = 2 * N_HALF
VOCAB = 100000
HIDDEN = 32
BATCH = 16384

NUM_CORES = 2
NUM_SUBCORES = 16
NUM_WORKERS = NUM_CORES * NUM_SUBCORES
CHUNK = BATCH // NUM_WORKERS
GATHER_W = 128
N_SUB = CHUNK // GATHER_W
NBUF = 4
DEPTH = NBUF - 1

QUART = 25088
VPAD = 4 * QUART
TCH = 3584
TROWS = QUART
ROWS128 = N_HALF * TROWS


def _transpose_body(in0, in1, in2, in3, out_ref):
    eye = jnp.eye(HIDDEN, dtype=jnp.float32)
    cols = [
        lax.dot_general(ref[0], eye, (((0,), (0,)), ((), ())),
                        preferred_element_type=jnp.float32)
        for ref in (in0, in1, in2, in3)
    ]
    out_ref[...] = jnp.concatenate(cols, axis=1)


def _to_vocab_major(tab_t):
    nk = QUART // TCH
    in_specs = [
        pl.BlockSpec((1, HIDDEN, TCH),
                     lambda t, k, p=p: (t, 0, p * nk + k))
        for p in range(4)
    ]
    return pl.pallas_call(
        _transpose_body,
        grid=(N_HALF, nk),
        in_specs=in_specs,
        out_specs=pl.BlockSpec((TCH, 128), lambda t, k: (t * nk + k, 0)),
        out_shape=jax.ShapeDtypeStruct((ROWS128, 128), jnp.float32),
    )(tab_t, tab_t, tab_t, tab_t)


def _body(user_ref, item_ref, ut_ref, it_ref, out_ref,
          idx_all, rows, sem_idx, sems_g, sems_s):
    wid = lax.axis_index("s") * NUM_CORES + lax.axis_index("c")
    base = wid * CHUNK

    idx_copies = []
    for t in range(N_TABLES):
        src_idx = user_ref if t < N_HALF else item_ref
        idx_copies.append(pltpu.async_copy(
            src_idx.at[t % N_HALF, pl.ds(base, CHUNK)],
            idx_all.at[pl.ds(t * CHUNK, CHUNK)],
            sem_idx))
    for c in idx_copies:
        c.wait()

    LANES = 16
    per_table = CHUNK // LANES

    def _fix(i, _):
        t = i // per_table
        off = lax.rem(t, N_HALF) * VPAD
        sl = pl.ds(i * LANES, LANES)
        v = idx_all[sl]
        q = (jnp.where(v >= QUART, 1, 0) + jnp.where(v >= 2 * QUART, 1, 0)
             + jnp.where(v >= 3 * QUART, 1, 0))
        idx_all[sl] = off + (v - q * QUART) * 4 + q
        return ()
    lax.fori_loop(0, N_TABLES * per_table, _fix, (), unroll=False)

    def fire_gathers(t):
        b = t % NBUF
        src_tab = ut_ref if t < N_HALF else it_ref
        cps = []
        for j in range(N_SUB):
            isl = pl.ds(t * CHUNK + j * GATHER_W, GATHER_W)
            rsl = pl.ds(j * GATHER_W, GATHER_W)
            cps.append(pltpu.async_copy(
                src_tab.at[idx_all.at[isl]], rows.at[b, rsl], sems_g[b]))
        return cps

    def fire_store(t):
        b = t % NBUF
        return pltpu.async_copy(
            rows.at[b],
            out_ref.at[pl.ds(base, CHUNK), pl.ds(t * HIDDEN, HIDDEN)],
            sems_s[b])

    g = [None] * N_TABLES
    s = [None] * N_TABLES
    for t in range(DEPTH):
        g[t] = fire_gathers(t)
    for t in range(N_TABLES):
        if t + DEPTH < N_TABLES:
            if t >= 1:
                s[t - 1].wait()
            g[t + DEPTH] = fire_gathers(t + DEPTH)
        for c in g[t]:
            c.wait()
        s[t] = fire_store(t)
    for t in range(N_TABLES - DEPTH - 1, N_TABLES):
        if s[t] is not None:
            s[t].wait()


@jax.jit
def _run(user, item_cat, ut1d, it1d):
    mesh = plsc.VectorSubcoreMesh(
        core_axis_name="c", subcore_axis_name="s",
        num_cores=NUM_CORES, num_subcores=NUM_SUBCORES,
    )
    k = pl.kernel(
        _body,
        out_type=jax.ShapeDtypeStruct((BATCH, N_TABLES * HIDDEN), jnp.float32),
        mesh=mesh,
        scratch_types=[
            pltpu.VMEM((N_TABLES * CHUNK,), jnp.int32),
            pltpu.VMEM((NBUF, CHUNK, HIDDEN), jnp.float32),
            pltpu.SemaphoreType.DMA,
            [pltpu.SemaphoreType.DMA] * NBUF,
            [pltpu.SemaphoreType.DMA] * NBUF,
        ],
        compiler_params=pltpu.CompilerParams(use_tc_tiling_on_sc=False),
    )
    return k(user, item_cat, ut1d, it1d)


def kernel(user, item_cat, item_con, user_tables, item_tables):
    del item_con
    ut = _to_vocab_major(user_tables.transpose(0, 2, 1))
    it = _to_vocab_major(item_tables.transpose(0, 2, 1))
    ut = ut.reshape(N_HALF * VPAD, HIDDEN)
    it = it.reshape(N_HALF * VPAD, HIDDEN)
    return jax.jit(_run)(user, item_cat, ut, it)


# speedup vs baseline: 1.8516x; 1.8516x over previous
import jax
import jax.numpy as jnp
from jax import lax
from jax.experimental import pallas as pl
from jax.experimental.pallas import tpu as pltpu
from jax.experimental.pallas import tpu_sc as plsc

N_HALF = 13
N_TABLES = 2 * N_HALF
VOCAB = 100000
HIDDEN = 32
BATCH = 16384

NUM_CORES = 2
NUM_SUBCORES = 16
NUM_WORKERS = NUM_CORES * NUM_SUBCORES
CHUNK = BATCH // NUM_WORKERS
GATHER_W = 128
N_SUB = CHUNK // GATHER_W
NBUF = 4
DEPTH = NBUF - 1

QUART = 25088
VPAD = 4 * QUART
TCH = 3584
TROWS = QUART
ROWS128 = N_HALF * TROWS


def _transpose_body(in0, in1, in2, in3, out_ref):
    s = jnp.concatenate([ref[0] for ref in (in0, in1, in2, in3)], axis=0)
    eye = jnp.eye(128, dtype=jnp.float32)
    out_ref[...] = lax.dot_general(s, eye, (((0,), (0,)), ((), ())),
                                   preferred_element_type=jnp.float32)


def _to_vocab_major(tab_t):
    nk = QUART // TCH
    in_specs = [
        pl.BlockSpec((1, HIDDEN, TCH),
                     lambda t, k, p=p: (t, 0, p * nk + k))
        for p in range(4)
    ]
    return pl.pallas_call(
        _transpose_body,
        grid=(N_HALF, nk),
        in_specs=in_specs,
        out_specs=pl.BlockSpec((TCH, 128), lambda t, k: (t * nk + k, 0)),
        out_shape=jax.ShapeDtypeStruct((ROWS128, 128), jnp.float32),
    )(tab_t, tab_t, tab_t, tab_t)


def _body(user_ref, item_ref, ut_ref, it_ref, out_ref,
          idx_all, rows, sem_idx, sems_g, sems_s):
    wid = lax.axis_index("s") * NUM_CORES + lax.axis_index("c")
    base = wid * CHUNK

    idx_copies = []
    for t in range(N_TABLES):
        src_idx = user_ref if t < N_HALF else item_ref
        idx_copies.append(pltpu.async_copy(
            src_idx.at[t % N_HALF, pl.ds(base, CHUNK)],
            idx_all.at[pl.ds(t * CHUNK, CHUNK)],
            sem_idx))
    for c in idx_copies:
        c.wait()

    LANES = 16
    per_table = CHUNK // LANES

    def _fix(i, _):
        t = i // per_table
        off = lax.rem(t, N_HALF) * VPAD
        sl = pl.ds(i * LANES, LANES)
        v = idx_all[sl]
        q = (jnp.where(v >= QUART, 1, 0) + jnp.where(v >= 2 * QUART, 1, 0)
             + jnp.where(v >= 3 * QUART, 1, 0))
        idx_all[sl] = off + (v - q * QUART) * 4 + q
        return ()
    lax.fori_loop(0, N_TABLES * per_table, _fix, (), unroll=False)

    def fire_gathers(t):
        b = t % NBUF
        src_tab = ut_ref if t < N_HALF else it_ref
        cps = []
        for j in range(N_SUB):
            isl = pl.ds(t * CHUNK + j * GATHER_W, GATHER_W)
            rsl = pl.ds(j * GATHER_W, GATHER_W)
            cps.append(pltpu.async_copy(
                src_tab.at[idx_all.at[isl]], rows.at[b, rsl], sems_g[b]))
        return cps

    def fire_store(t):
        b = t % NBUF
        return pltpu.async_copy(
            rows.at[b],
            out_ref.at[pl.ds(base, CHUNK), pl.ds(t * HIDDEN, HIDDEN)],
            sems_s[b])

    g = [None] * N_TABLES
    s = [None] * N_TABLES
    for t in range(DEPTH):
        g[t] = fire_gathers(t)
    for t in range(N_TABLES):
        if t + DEPTH < N_TABLES:
            if t >= 1:
                s[t - 1].wait()
            g[t + DEPTH] = fire_gathers(t + DEPTH)
        for c in g[t]:
            c.wait()
        s[t] = fire_store(t)
    for t in range(N_TABLES - DEPTH - 1, N_TABLES):
        if s[t] is not None:
            s[t].wait()


@jax.jit
def _run(user, item_cat, ut1d, it1d):
    mesh = plsc.VectorSubcoreMesh(
        core_axis_name="c", subcore_axis_name="s",
        num_cores=NUM_CORES, num_subcores=NUM_SUBCORES,
    )
    k = pl.kernel(
        _body,
        out_type=jax.ShapeDtypeStruct((BATCH, N_TABLES * HIDDEN), jnp.float32),
        mesh=mesh,
        scratch_types=[
            pltpu.VMEM((N_TABLES * CHUNK,), jnp.int32),
            pltpu.VMEM((NBUF, CHUNK, HIDDEN), jnp.float32),
            pltpu.SemaphoreType.DMA,
            [pltpu.SemaphoreType.DMA] * NBUF,
            [pltpu.SemaphoreType.DMA] * NBUF,
        ],
        compiler_params=pltpu.CompilerParams(use_tc_tiling_on_sc=False),
    )
    return k(user, item_cat, ut1d, it1d)


def kernel(user, item_cat, item_con, user_tables, item_tables):
    del item_con
    ut = _to_vocab_major(user_tables.transpose(0, 2, 1))
    it = _to_vocab_major(item_tables.transpose(0, 2, 1))
    ut = ut.reshape(N_HALF * VPAD, HIDDEN)
    it = it.reshape(N_HALF * VPAD, HIDDEN)
    return jax.jit(_run)(user, item_cat, ut, it)
